# baseline (speedup 1.0000x reference)
import functools

import jax
import jax.numpy as jnp
from jax import lax
from jax.experimental import pallas as pl
from jax.experimental.pallas import tpu as pltpu
from jax.experimental.pallas import tpu_sc as plsc

N_NODES = 10000
N_EDGES = 320000
F = 128

NC = 2
NS = 16
L = 16
NW = NC * NS
CH = 128
CPW = 80
EPW = CPW * CH
E_PAD = NW * EPW

_SLABS = [(t * 624, 624) for t in range(NS - 1)] + [((NS - 1) * 624, 640)]


def _slab_chunks(off, ln):
    out = []
    r = 0
    while r < ln:
        n = min(CH, ln - r)
        out.append((off + r, n))
        r += n
    return out


def _sc_aggregate(x, edata, ew):
    mesh = plsc.VectorSubcoreMesh(core_axis_name="c", subcore_axis_name="s")

    @functools.partial(
        pl.kernel,
        out_type=[jax.ShapeDtypeStruct((NC, N_NODES, F), jnp.float32),
                  jax.ShapeDtypeStruct((E_PAD, F), jnp.float32)],
        mesh=mesh,
        scratch_types=[
            pltpu.VMEM((CPW, 2, CH), jnp.int32),
            pltpu.VMEM((CPW, CH), jnp.float32),
            pltpu.VMEM((CH, F), jnp.float32),
            pltpu.VMEM_SHARED((N_NODES, F), jnp.float32),
            pltpu.SemaphoreType.DMA,
        ],
    )
    def body(x_hbm, ed_hbm, ew_hbm, out_hbm, msgs_hbm,
             ed_v, w_v, rows_v, spx, sem):
        cid = lax.axis_index("c")
        sid = lax.axis_index("s")
        wid = cid * NS + sid

        for t, (off, ln) in enumerate(_SLABS):
            @pl.when(sid == t)
            def _():
                for o, n in _slab_chunks(off, ln):
                    pltpu.sync_copy(x_hbm.at[pl.ds(o, n)],
                                    spx.at[pl.ds(o, n)])
        pltpu.sync_copy(ed_hbm.at[wid], ed_v)
        pltpu.sync_copy(ew_hbm.at[wid], w_v)
        plsc.subcore_barrier()

        dnums = lax.GatherDimensionNumbers(
            offset_dims=(), collapsed_slice_dims=(0,), start_index_map=(0,))
        base = wid * EPW

        def produce(j, carry):
            pltpu.async_copy(spx.at[ed_v.at[j, 0]], rows_v, sem).wait()

            def scale16(k, c2):
                wv = w_v[j, pl.ds(pl.multiple_of(k * L, L), L)]
                for e in range(L):
                    s = lax.gather(wv, jnp.full((L, 1), e, jnp.int32),
                                   dnums, slice_sizes=(1,),
                                   mode=lax.GatherScatterMode.PROMISE_IN_BOUNDS)
                    row = k * L + e
                    for q in range(F // L):
                        sl = pl.ds(q * L, L)
                        rows_v[row, sl] = rows_v[row, sl] * s
                return c2

            lax.fori_loop(0, CH // L, scale16, 0)
            pltpu.sync_copy(
                rows_v,
                msgs_hbm.at[pl.ds(base + pl.multiple_of(j * CH, CH), CH)])
            return carry

        lax.fori_loop(0, CPW, produce, 0)
        plsc.subcore_barrier()

        zero = jnp.zeros((L,), jnp.float32)

        def zrow(i, carry):
            for q in range(F // L):
                rows_v[i, pl.ds(q * L, L)] = zero
            return carry

        lax.fori_loop(0, CH, zrow, 0)

        for t, (off, ln) in enumerate(_SLABS):
            @pl.when(sid == t)
            def _():
                for o, n in _slab_chunks(off, ln):
                    pltpu.sync_copy(rows_v.at[pl.ds(0, n)],
                                    spx.at[pl.ds(o, n)])
        plsc.subcore_barrier()

        def consume(j, carry):
            pltpu.sync_copy(
                msgs_hbm.at[pl.ds(base + pl.multiple_of(j * CH, CH), CH)],
                rows_v)
            pltpu.sync_copy(rows_v, spx.at[ed_v.at[j, 1]], add=True)
            return carry

        lax.fori_loop(0, CPW, consume, 0)
        plsc.subcore_barrier()

        for t, (off, ln) in enumerate(_SLABS):
            @pl.when(sid == t)
            def _():
                for o, n in _slab_chunks(off, ln):
                    pltpu.sync_copy(spx.at[pl.ds(o, n)],
                                    rows_v.at[pl.ds(0, n)])
                    pltpu.sync_copy(rows_v.at[pl.ds(0, n)],
                                    out_hbm.at[cid, pl.ds(o, n)])

    return body(x, edata, ew)


def _tc_combine_matmul(y, W, b):
    blk = 1000

    def body(y_ref, w_ref, b_ref, o_ref):
        ys = y_ref[0] + y_ref[1]
        o_ref[...] = (jnp.dot(ys, w_ref[...], preferred_element_type=jnp.float32)
                      + b_ref[...])

    return pl.pallas_call(
        body,
        grid=(N_NODES // blk,),
        in_specs=[
            pl.BlockSpec((NC, blk, F), lambda i: (0, i, 0)),
            pl.BlockSpec((F, F), lambda i: (0, 0)),
            pl.BlockSpec((1, F), lambda i: (0, 0)),
        ],
        out_specs=pl.BlockSpec((blk, F), lambda i: (i, 0)),
        out_shape=jax.ShapeDtypeStruct((N_NODES, F), jnp.float32),
    )(y, W, b.reshape(1, F))


def kernel(x, edge_index, edge_weight, W, b):
    src = edge_index[0].astype(jnp.int32)
    dst = edge_index[1].astype(jnp.int32)

    pad = E_PAD - N_EDGES
    edata = jnp.stack([
        jnp.pad(src, (0, pad)),
        jnp.pad(dst, (0, pad)),
    ], axis=0).reshape(2, NW, CPW, CH).transpose(1, 2, 0, 3)
    ew = jnp.pad(edge_weight.astype(jnp.float32),
                 (0, pad)).reshape(NW, CPW, CH)

    y, _ = _sc_aggregate(x, edata, ew)
    return _tc_combine_matmul(y, W, b)


# speedup vs baseline: 1.3485x; 1.3485x over previous
import functools

import jax
import jax.numpy as jnp
from jax import lax
from jax.experimental import pallas as pl
from jax.experimental.pallas import tpu as pltpu
from jax.experimental.pallas import tpu_sc as plsc

N_NODES = 10000
N_EDGES = 320000
F = 128

NC = 2
NS = 16
L = 16
NW = NC * NS
CH = 128
CPW = 80
OUTER = CPW // 2
EPW = CPW * CH
E_PAD = NW * EPW

_SLABS = [(t * 624, 624) for t in range(NS - 1)] + [((NS - 1) * 624, 640)]


def _slab_chunks(off, ln):
    out = []
    r = 0
    while r < ln:
        n = min(CH, ln - r)
        out.append((off + r, n))
        r += n
    return out


def _sc_aggregate(x, esrc, edst, ew):
    mesh = plsc.VectorSubcoreMesh(core_axis_name="c", subcore_axis_name="s")

    @functools.partial(
        pl.kernel,
        out_type=[jax.ShapeDtypeStruct((NC, N_NODES, F), jnp.float32),
                  jax.ShapeDtypeStruct((E_PAD, F), jnp.float32)],
        mesh=mesh,
        scratch_types=[
            pltpu.VMEM((CPW, CH), jnp.int32),
            pltpu.VMEM((CPW // 2, CH), jnp.float32),
            pltpu.VMEM((2, CH, F), jnp.float32),
            pltpu.VMEM_SHARED((N_NODES, F), jnp.float32),
            [pltpu.SemaphoreType.DMA] * 2,
            [pltpu.SemaphoreType.DMA] * 2,
        ],
    )
    def body(x_hbm, src_hbm, dst_hbm, ew_hbm, out_hbm, msgs_hbm,
             idx_v, w_v, rows_v, spx, isems, osems):
        cid = lax.axis_index("c")
        sid = lax.axis_index("s")
        wid = cid * NS + sid

        for t, (off, ln) in enumerate(_SLABS):
            @pl.when(sid == t)
            def _():
                for o, n in _slab_chunks(off, ln):
                    pltpu.sync_copy(x_hbm.at[pl.ds(o, n)],
                                    spx.at[pl.ds(o, n)])
        pltpu.sync_copy(src_hbm.at[wid], idx_v)
        pltpu.sync_copy(ew_hbm.at[wid, pl.ds(0, CPW // 2)], w_v)
        plsc.subcore_barrier()

        dnums = lax.GatherDimensionNumbers(
            offset_dims=(), collapsed_slice_dims=(0,), start_index_map=(0,))
        base = wid * EPW

        def fire_gather(j, u):
            pltpu.async_copy(spx.at[idx_v.at[j]], rows_v.at[u], isems[u])

        def wait_gather(j, u):
            pltpu.make_async_copy(spx.at[idx_v.at[j]], rows_v.at[u],
                                  isems[u]).wait()

        def msgs_ref(j):
            return msgs_hbm.at[pl.ds(base + pl.multiple_of(j * CH, CH), CH)]

        def fire_write(j, u):
            pltpu.async_copy(rows_v.at[u], msgs_ref(j), osems[u])

        def wait_write(j, u):
            pltpu.make_async_copy(rows_v.at[u], msgs_ref(j), osems[u]).wait()

        def scale(j, u):
            wrow = jnp.where(j >= CPW // 2, j - CPW // 2, j)

            def scale16(k, c2):
                wv = w_v[wrow, pl.ds(pl.multiple_of(k * L, L), L)]
                for e in range(L):
                    s = lax.gather(wv, jnp.full((L, 1), e, jnp.int32),
                                   dnums, slice_sizes=(1,),
                                   mode=lax.GatherScatterMode.PROMISE_IN_BOUNDS)
                    row = k * L + e
                    for q in range(F // L):
                        sl = pl.ds(q * L, L)
                        rows_v[u, row, sl] = rows_v[u, row, sl] * s
                return c2

            lax.fori_loop(0, CH // L, scale16, 0)

        fire_gather(0, 0)

        def produce(i, carry):
            for u in range(2):
                j = i * 2 + u
                wait_gather(j, u)
                scale(j, u)
                if u == 0:
                    @pl.when(i > 0)
                    def _():
                        wait_write(j - 1, 1 - u)
                    fire_gather(j + 1, 1 - u)
                else:
                    @pl.when(i < OUTER - 1)
                    def _():
                        wait_write(j - 1, 1 - u)
                        fire_gather(j + 1, 1 - u)
                fire_write(j, u)

            @pl.when(i == OUTER // 2 - 1)
            def _():
                pltpu.sync_copy(ew_hbm.at[wid, pl.ds(CPW // 2, CPW // 2)],
                                w_v)
            return carry

        lax.fori_loop(0, OUTER, produce, 0)
        wait_write(CPW - 2, 0)
        wait_write(CPW - 1, 1)

        pltpu.sync_copy(dst_hbm.at[wid], idx_v)
        plsc.subcore_barrier()

        zero = jnp.zeros((L,), jnp.float32)

        def zrow(i, carry):
            for q in range(F // L):
                rows_v[0, i, pl.ds(q * L, L)] = zero
            return carry

        lax.fori_loop(0, CH, zrow, 0)

        for t, (off, ln) in enumerate(_SLABS):
            @pl.when(sid == t)
            def _():
                for o, n in _slab_chunks(off, ln):
                    pltpu.sync_copy(rows_v.at[0, pl.ds(0, n)],
                                    spx.at[pl.ds(o, n)])
        plsc.subcore_barrier()

        def fire_read(j, u):
            pltpu.async_copy(msgs_ref(j), rows_v.at[u], isems[u])

        def wait_read(j, u):
            pltpu.make_async_copy(msgs_ref(j), rows_v.at[u], isems[u]).wait()

        def fire_scatter(j, u):
            pltpu.async_copy(rows_v.at[u], spx.at[idx_v.at[j]],
                             osems[u], add=True)

        def wait_scatter(j, u):
            pltpu.make_async_copy(rows_v.at[u], spx.at[idx_v.at[j]],
                                  osems[u]).wait()

        fire_read(0, 0)

        def consume(i, carry):
            for u in range(2):
                j = i * 2 + u
                wait_read(j, u)
                if u == 0:
                    @pl.when(i > 0)
                    def _():
                        wait_scatter(j - 1, 1 - u)
                    fire_read(j + 1, 1 - u)
                else:
                    @pl.when(i < OUTER - 1)
                    def _():
                        wait_scatter(j - 1, 1 - u)
                        fire_read(j + 1, 1 - u)
                fire_scatter(j, u)
            return carry

        lax.fori_loop(0, OUTER, consume, 0)
        wait_scatter(CPW - 2, 0)
        wait_scatter(CPW - 1, 1)
        plsc.subcore_barrier()

        for t, (off, ln) in enumerate(_SLABS):
            @pl.when(sid == t)
            def _():
                for o, n in _slab_chunks(off, ln):
                    pltpu.sync_copy(spx.at[pl.ds(o, n)],
                                    rows_v.at[0, pl.ds(0, n)])
                    pltpu.sync_copy(rows_v.at[0, pl.ds(0, n)],
                                    out_hbm.at[cid, pl.ds(o, n)])

    return body(x, esrc, edst, ew)


def _tc_combine_matmul(y, W, b):
    blk = 1000

    def body(y_ref, w_ref, b_ref, o_ref):
        ys = y_ref[0] + y_ref[1]
        o_ref[...] = (jnp.dot(ys, w_ref[...], preferred_element_type=jnp.float32)
                      + b_ref[...])

    return pl.pallas_call(
        body,
        grid=(N_NODES // blk,),
        in_specs=[
            pl.BlockSpec((NC, blk, F), lambda i: (0, i, 0)),
            pl.BlockSpec((F, F), lambda i: (0, 0)),
            pl.BlockSpec((1, F), lambda i: (0, 0)),
        ],
        out_specs=pl.BlockSpec((blk, F), lambda i: (i, 0)),
        out_shape=jax.ShapeDtypeStruct((N_NODES, F), jnp.float32),
    )(y, W, b.reshape(1, F))


def kernel(x, edge_index, edge_weight, W, b):
    src = edge_index[0].astype(jnp.int32)
    dst = edge_index[1].astype(jnp.int32)

    pad = E_PAD - N_EDGES
    esrc = jnp.pad(src, (0, pad)).reshape(NW, CPW, CH)
    edst = jnp.pad(dst, (0, pad)).reshape(NW, CPW, CH)
    ew = jnp.pad(edge_weight.astype(jnp.float32),
                 (0, pad)).reshape(NW, CPW, CH)

    y, _ = _sc_aggregate(x, esrc, edst, ew)
    return _tc_combine_matmul(y, W, b)


# speedup vs baseline: 1.3665x; 1.0133x over previous
import functools

import jax
import jax.numpy as jnp
from jax import lax
from jax.experimental import pallas as pl
from jax.experimental.pallas import tpu as pltpu
from jax.experimental.pallas import tpu_sc as plsc

N_NODES = 10000
N_EDGES = 320000
F = 128

NC = 2
NS = 16
L = 16
NW = NC * NS
CH = 128
CPW = 80
OUTER = CPW // 2
EPW = CPW * CH
E_PAD = NW * EPW

_SLABS = [(t * 624, 624) for t in range(NS - 1)] + [((NS - 1) * 624, 640)]


def _slab_chunks(off, ln):
    out = []
    r = 0
    while r < ln:
        n = min(CH, ln - r)
        out.append((off + r, n))
        r += n
    return out


def _sc_aggregate(x, esrc, edst, ew):
    mesh = plsc.VectorSubcoreMesh(core_axis_name="c", subcore_axis_name="s")

    @functools.partial(
        pl.kernel,
        out_type=[jax.ShapeDtypeStruct((NC, N_NODES, F), jnp.float32),
                  jax.ShapeDtypeStruct((E_PAD, F), jnp.float32)],
        mesh=mesh,
        scratch_types=[
            pltpu.VMEM((CPW, CH), jnp.int32),
            pltpu.VMEM((CPW // 2, CH), jnp.float32),
            pltpu.VMEM((2, CH, F), jnp.float32),
            pltpu.VMEM_SHARED((N_NODES, F), jnp.float32),
            [pltpu.SemaphoreType.DMA] * 2,
            [pltpu.SemaphoreType.DMA] * 2,
        ],
    )
    def body(x_hbm, src_hbm, dst_hbm, ew_hbm, out_hbm, msgs_hbm,
             idx_v, w_v, rows_v, spx, isems, osems):
        cid = lax.axis_index("c")
        sid = lax.axis_index("s")
        wid = cid * NS + sid

        for t, (off, ln) in enumerate(_SLABS):
            @pl.when(sid == t)
            def _():
                for o, n in _slab_chunks(off, ln):
                    pltpu.sync_copy(x_hbm.at[pl.ds(o, n)],
                                    spx.at[pl.ds(o, n)])
        pltpu.sync_copy(src_hbm.at[wid], idx_v)
        pltpu.sync_copy(ew_hbm.at[wid, pl.ds(0, CPW // 2)], w_v)
        plsc.subcore_barrier()

        dnums = lax.GatherDimensionNumbers(
            offset_dims=(), collapsed_slice_dims=(0,), start_index_map=(0,))
        base = wid * EPW

        def fire_gather(j, u):
            pltpu.async_copy(spx.at[idx_v.at[j]], rows_v.at[u], isems[u])

        def wait_gather(j, u):
            pltpu.make_async_copy(spx.at[idx_v.at[j]], rows_v.at[u],
                                  isems[u]).wait()

        def msgs_ref(j):
            return msgs_hbm.at[pl.ds(base + pl.multiple_of(j * CH, CH), CH)]

        def fire_write(j, u):
            pltpu.async_copy(rows_v.at[u], msgs_ref(j), osems[u])

        def wait_write(j, u):
            pltpu.make_async_copy(rows_v.at[u], msgs_ref(j), osems[u]).wait()

        def scale(j, u):
            wrow = jnp.where(j >= CPW // 2, j - CPW // 2, j)

            def scale16(k, c2):
                wv = w_v[wrow, pl.ds(pl.multiple_of(k * L, L), L)]
                for e in range(L):
                    s = lax.gather(wv, jnp.full((L, 1), e, jnp.int32),
                                   dnums, slice_sizes=(1,),
                                   mode=lax.GatherScatterMode.PROMISE_IN_BOUNDS)
                    row = k * L + e
                    for q in range(F // L):
                        sl = pl.ds(q * L, L)
                        rows_v[u, row, sl] = rows_v[u, row, sl] * s
                return c2

            lax.fori_loop(0, CH // L, scale16, 0)

        fire_gather(0, 0)

        def produce(i, carry):
            for u in range(2):
                j = i * 2 + u
                wait_gather(j, u)
                if u == 0:
                    @pl.when(i > 0)
                    def _():
                        wait_write(j - 1, 1 - u)
                    fire_gather(j + 1, 1 - u)
                else:
                    @pl.when(i < OUTER - 1)
                    def _():
                        wait_write(j - 1, 1 - u)
                        fire_gather(j + 1, 1 - u)
                scale(j, u)
                fire_write(j, u)

            @pl.when(i == OUTER // 2 - 1)
            def _():
                pltpu.sync_copy(ew_hbm.at[wid, pl.ds(CPW // 2, CPW // 2)],
                                w_v)
            return carry

        lax.fori_loop(0, OUTER, produce, 0)
        wait_write(CPW - 2, 0)
        wait_write(CPW - 1, 1)

        pltpu.sync_copy(dst_hbm.at[wid], idx_v)
        plsc.subcore_barrier()

        zero = jnp.zeros((L,), jnp.float32)

        def zrow(i, carry):
            for q in range(F // L):
                rows_v[0, i, pl.ds(q * L, L)] = zero
            return carry

        lax.fori_loop(0, CH, zrow, 0)

        for t, (off, ln) in enumerate(_SLABS):
            @pl.when(sid == t)
            def _():
                for o, n in _slab_chunks(off, ln):
                    pltpu.sync_copy(rows_v.at[0, pl.ds(0, n)],
                                    spx.at[pl.ds(o, n)])
        plsc.subcore_barrier()

        def fire_read(j, u):
            pltpu.async_copy(msgs_ref(j), rows_v.at[u], isems[u])

        def wait_read(j, u):
            pltpu.make_async_copy(msgs_ref(j), rows_v.at[u], isems[u]).wait()

        def fire_scatter(j, u):
            pltpu.async_copy(rows_v.at[u], spx.at[idx_v.at[j]],
                             osems[u], add=True)

        def wait_scatter(j, u):
            pltpu.make_async_copy(rows_v.at[u], spx.at[idx_v.at[j]],
                                  osems[u]).wait()

        fire_read(0, 0)

        def consume(i, carry):
            for u in range(2):
                j = i * 2 + u
                wait_read(j, u)
                if u == 0:
                    @pl.when(i > 0)
                    def _():
                        wait_scatter(j - 1, 1 - u)
                    fire_read(j + 1, 1 - u)
                else:
                    @pl.when(i < OUTER - 1)
                    def _():
                        wait_scatter(j - 1, 1 - u)
                        fire_read(j + 1, 1 - u)
                fire_scatter(j, u)
            return carry

        lax.fori_loop(0, OUTER, consume, 0)
        wait_scatter(CPW - 2, 0)
        wait_scatter(CPW - 1, 1)
        plsc.subcore_barrier()

        for t, (off, ln) in enumerate(_SLABS):
            @pl.when(sid == t)
            def _():
                for o, n in _slab_chunks(off, ln):
                    pltpu.sync_copy(spx.at[pl.ds(o, n)],
                                    rows_v.at[0, pl.ds(0, n)])
                    pltpu.sync_copy(rows_v.at[0, pl.ds(0, n)],
                                    out_hbm.at[cid, pl.ds(o, n)])

    return body(x, esrc, edst, ew)


def _tc_combine_matmul(y, W, b):
    blk = 1000

    def body(y_ref, w_ref, b_ref, o_ref):
        ys = y_ref[0] + y_ref[1]
        o_ref[...] = (jnp.dot(ys, w_ref[...], preferred_element_type=jnp.float32)
                      + b_ref[...])

    return pl.pallas_call(
        body,
        grid=(N_NODES // blk,),
        in_specs=[
            pl.BlockSpec((NC, blk, F), lambda i: (0, i, 0)),
            pl.BlockSpec((F, F), lambda i: (0, 0)),
            pl.BlockSpec((1, F), lambda i: (0, 0)),
        ],
        out_specs=pl.BlockSpec((blk, F), lambda i: (i, 0)),
        out_shape=jax.ShapeDtypeStruct((N_NODES, F), jnp.float32),
    )(y, W, b.reshape(1, F))


def kernel(x, edge_index, edge_weight, W, b):
    src = edge_index[0].astype(jnp.int32)
    dst = edge_index[1].astype(jnp.int32)

    pad = E_PAD - N_EDGES
    esrc = jnp.pad(src, (0, pad)).reshape(NW, CPW, CH)
    edst = jnp.pad(dst, (0, pad)).reshape(NW, CPW, CH)
    ew = jnp.pad(edge_weight.astype(jnp.float32),
                 (0, pad)).reshape(NW, CPW, CH)

    y, _ = _sc_aggregate(x, esrc, edst, ew)
    return _tc_combine_matmul(y, W, b)
